# baseline (speedup 1.0000x reference)
import jax
import jax.numpy as jnp
from jax import lax
from jax.experimental import pallas as pl
from jax.experimental.pallas import tpu as pltpu
from jax.experimental.pallas import tpu_sc as plsc

_B, _C, _H, _W = 128, 10, 128, 128
_NC, _NS, _L = 2, 16, 16
_NWORK = _NC * _NS
_BPW = _B // _NWORK
_SLAB = 2
_NSLAB = _C // _SLAB


def _mesh():
    return plsc.VectorSubcoreMesh(core_axis_name="c", subcore_axis_name="s")


def _row_sum(buf, r):
    z = jnp.zeros((_L,), jnp.float32)

    def body(hh, accs):
        a0, a1, a2, a3 = accs
        h = hh * 4
        for hi in (0, 1, 2, 3):
            a0 = a0 + buf[r, h + hi, pl.ds(0, _L)] + buf[r, h + hi, pl.ds(64, _L)]
            a1 = a1 + buf[r, h + hi, pl.ds(16, _L)] + buf[r, h + hi, pl.ds(80, _L)]
            a2 = a2 + buf[r, h + hi, pl.ds(32, _L)] + buf[r, h + hi, pl.ds(96, _L)]
            a3 = a3 + buf[r, h + hi, pl.ds(48, _L)] + buf[r, h + hi, pl.ds(112, _L)]
        return a0, a1, a2, a3

    a0, a1, a2, a3 = lax.fori_loop(0, _H // 4, body, (z, z, z, z))
    return jnp.sum((a0 + a1) + (a2 + a3))


def _fixup_planes(fbg, ftgt, bgval):
    def body(h, carry):
        for k in range(_W // _L):
            g = fbg[h, pl.ds(k * _L, _L)]
            t = ftgt[h, pl.ds(k * _L, _L)]
            m = g > 0.5
            fbg[h, pl.ds(k * _L, _L)] = jnp.where(m, bgval, g)
            ftgt[h, pl.ds(k * _L, _L)] = jnp.where(m, 1.0, t)
        return carry

    lax.fori_loop(0, _H, body, 0)


def _sc_body(g_ref, t_ref, out_ref,
             buf_a, buf_b, fbg, ftgt, tbuf,
             rs_a, rs_b, ws_a, ws_b, fs_a, fs_b, fw_a, fw_b):
    cid = lax.axis_index("c")
    sid = lax.axis_index("s")
    wid = sid * _NC + cid
    pltpu.sync_copy(t_ref, tbuf)
    tgt = jnp.max(tbuf[...]).astype(jnp.int32)

    bufs = (buf_a, buf_b)
    rsem = (rs_a, rs_b)
    wsem = (ws_a, ws_b)

    fix_pending = None
    fix_writes = None

    def run_fixup(pending, writes):
        b_p, bgc_p, gb, gt = pending
        if writes is not None:
            writes[0].wait()
            writes[1].wait()
        gb.wait()
        gt.wait()
        bgval = jnp.where(bgc_p == tgt, 1.0, 0.0)
        _fixup_planes(fbg, ftgt, bgval)
        wb = pltpu.async_copy(fbg, out_ref.at[b_p, bgc_p], fw_a)
        wt = pltpu.async_copy(ftgt, out_ref.at[b_p, tgt], fw_b)
        return (wb, wt)

    for j in range(_BPW):
        b = wid * _BPW + j
        rcp = [None] * _NSLAB
        wcp = [None] * _NSLAB
        rcp[0] = pltpu.async_copy(
            g_ref.at[b, pl.ds(0, _SLAB)], bufs[0], rsem[0])
        if fix_pending is not None:
            fix_writes = run_fixup(fix_pending, fix_writes)
            fix_pending = None
        best = jnp.float32(0.0)
        bgc = jnp.int32(0)
        for s in range(_NSLAB):
            pbuf = bufs[s % 2]
            if s + 1 < _NSLAB:
                nxt = (s + 1) % 2
                if s >= 1:
                    wcp[s - 1].wait()
                rcp[s + 1] = pltpu.async_copy(
                    g_ref.at[b, pl.ds((s + 1) * _SLAB, _SLAB)],
                    bufs[nxt], rsem[nxt])
            rcp[s].wait()
            wcp[s] = pltpu.async_copy(
                pbuf, out_ref.at[b, pl.ds(s * _SLAB, _SLAB)], wsem[s % 2])
            for r in range(_SLAB):
                c = s * _SLAB + r
                sc = _row_sum(pbuf, r)
                if c == 0:
                    best = sc
                else:
                    upd = sc > best
                    best = jnp.where(upd, sc, best)
                    bgc = jnp.where(upd, jnp.int32(c), bgc)
        wcp[_NSLAB - 2].wait()
        wcp[_NSLAB - 1].wait()
        gb = pltpu.async_copy(g_ref.at[b, bgc], fbg, fs_a)
        gt = pltpu.async_copy(g_ref.at[b, tgt], ftgt, fs_b)
        fix_pending = (b, bgc, gb, gt)

    fix_writes = run_fixup(fix_pending, fix_writes)
    fix_writes[0].wait()
    fix_writes[1].wait()


def kernel(grid, target_color):
    tgt16 = jnp.full((_L,), target_color, jnp.float32)
    sck = pl.kernel(
        _sc_body,
        out_type=jax.ShapeDtypeStruct((_B, _C, _H, _W), jnp.float32),
        mesh=_mesh(),
        compiler_params=pltpu.CompilerParams(needs_layout_passes=False),
        scratch_types=[
            pltpu.VMEM((_SLAB, _H, _W), jnp.float32),
            pltpu.VMEM((_SLAB, _H, _W), jnp.float32),
            pltpu.VMEM((_H, _W), jnp.float32),
            pltpu.VMEM((_H, _W), jnp.float32),
            pltpu.VMEM((_L,), jnp.float32),
            pltpu.SemaphoreType.DMA,
            pltpu.SemaphoreType.DMA,
            pltpu.SemaphoreType.DMA,
            pltpu.SemaphoreType.DMA,
            pltpu.SemaphoreType.DMA,
            pltpu.SemaphoreType.DMA,
            pltpu.SemaphoreType.DMA,
            pltpu.SemaphoreType.DMA,
        ],
    )
    return sck(grid, tgt16)


# speedup vs baseline: 1.1395x; 1.1395x over previous
import jax
import jax.numpy as jnp
from jax import lax
from jax.experimental import pallas as pl
from jax.experimental.pallas import tpu as pltpu
from jax.experimental.pallas import tpu_sc as plsc

_B, _C, _H, _W = 128, 10, 128, 128
_NB = 16
_NC, _NS, _L = 2, 16, 16
_NWORK = _NC * _NS
_BPW = _B // _NWORK


def _sc_mesh():
    return plsc.VectorSubcoreMesh(core_axis_name="c", subcore_axis_name="s")


def _tc_body(g_ref, out_ref, bg_ref):
    out_ref[...] = g_ref[...]
    for i in range(_NB):
        g = g_ref[i]
        s2 = jnp.sum(g, axis=1)
        sums = jnp.sum(s2, axis=1, keepdims=True)
        smax = jnp.max(sums)
        ci2 = lax.broadcasted_iota(jnp.int32, (_C, 1), 0)
        bg = jnp.min(jnp.where(sums == smax, ci2, _C))
        bg_ref[i] = jnp.full((1, _L), bg, jnp.float32)


def _tc_copy_sums(grid):
    return pl.pallas_call(
        _tc_body,
        grid=(_B // _NB,),
        in_specs=[pl.BlockSpec((_NB, _C, _H, _W), lambda b: (b, 0, 0, 0))],
        out_specs=[
            pl.BlockSpec((_NB, _C, _H, _W), lambda b: (b, 0, 0, 0)),
            pl.BlockSpec((_NB, 1, _L), lambda b: (b, 0, 0)),
        ],
        out_shape=[
            jax.ShapeDtypeStruct((_B, _C, _H, _W), jnp.float32),
            jax.ShapeDtypeStruct((_B, 1, _L), jnp.float32),
        ],
    )(grid)


def _fixup_planes(fbg, ftgt, bgval):
    def body(h, carry):
        for k in range(_W // _L):
            g = fbg[h, pl.ds(k * _L, _L)]
            t = ftgt[h, pl.ds(k * _L, _L)]
            m = g > 0.5
            fbg[h, pl.ds(k * _L, _L)] = jnp.where(m, bgval, g)
            ftgt[h, pl.ds(k * _L, _L)] = jnp.where(m, 1.0, t)
        return carry

    lax.fori_loop(0, _H, body, 0)


def _sc_body(g_ref, bg_ref, t_ref, out_ref,
             fbg, ftgt, ibuf, tbuf,
             gs_a, gs_b, ws_a, ws_b):
    cid = lax.axis_index("c")
    sid = lax.axis_index("s")
    wid = sid * _NC + cid
    pltpu.sync_copy(t_ref, tbuf)
    tgt = jnp.max(tbuf[...]).astype(jnp.int32)

    writes = None
    for j in range(_BPW):
        b = wid * _BPW + j
        pltpu.sync_copy(bg_ref.at[b, 0], ibuf)
        bgc = jnp.max(ibuf[...]).astype(jnp.int32)
        gb = pltpu.async_copy(g_ref.at[b, bgc], fbg, gs_a)
        gt = pltpu.async_copy(g_ref.at[b, tgt], ftgt, gs_b)
        if writes is not None:
            writes[0].wait()
            writes[1].wait()
        gb.wait()
        gt.wait()
        bgval = jnp.where(bgc == tgt, 1.0, 0.0)
        _fixup_planes(fbg, ftgt, bgval)
        wb = pltpu.async_copy(fbg, out_ref.at[b, bgc], ws_a)
        wt = pltpu.async_copy(ftgt, out_ref.at[b, tgt], ws_b)
        writes = (wb, wt)
    writes[0].wait()
    writes[1].wait()


def _sc_fixup(grid, bgv, tgt16, out_ref):
    sck = pl.kernel(
        _sc_body,
        out_type=(),
        mesh=_sc_mesh(),
        compiler_params=pltpu.CompilerParams(needs_layout_passes=False),
        scratch_types=[
            pltpu.VMEM((_H, _W), jnp.float32),
            pltpu.VMEM((_H, _W), jnp.float32),
            pltpu.VMEM((_L,), jnp.float32),
            pltpu.VMEM((_L,), jnp.float32),
            pltpu.SemaphoreType.DMA,
            pltpu.SemaphoreType.DMA,
            pltpu.SemaphoreType.DMA,
            pltpu.SemaphoreType.DMA,
        ],
    )
    sck(grid, bgv, tgt16, out_ref)


def kernel(grid, target_color):
    tgt16 = jnp.full((_L,), target_color, jnp.float32)
    out, bgv = _tc_copy_sums(grid)
    out_ref = jax.new_ref(out)
    _sc_fixup(grid, bgv, tgt16, out_ref)
    return out_ref[...]


# speedup vs baseline: 1.1862x; 1.0410x over previous
import jax
import jax.numpy as jnp
from jax import lax
from jax.experimental import pallas as pl
from jax.experimental.pallas import tpu as pltpu
from jax.experimental.pallas import tpu_sc as plsc

_B, _C, _H, _W = 128, 10, 128, 128
_NB = 16
_NC, _NS, _L = 2, 16, 16
_NWORK = _NC * _NS
_BPW = _B // _NWORK


def _sc_mesh():
    return plsc.VectorSubcoreMesh(core_axis_name="c", subcore_axis_name="s")


def _tc_body(g_ref, out_ref, bg_ref):
    out_ref[...] = g_ref[...]
    for i in range(_NB):
        g = g_ref[i]
        s2 = jnp.sum(g, axis=1)
        sums = jnp.sum(s2, axis=1, keepdims=True)
        smax = jnp.max(sums)
        ci2 = lax.broadcasted_iota(jnp.int32, (_C, 1), 0)
        bg = jnp.min(jnp.where(sums == smax, ci2, _C))
        bg_ref[i] = jnp.full((1, _L), bg, jnp.float32)


def _tc_copy_sums(grid):
    return pl.pallas_call(
        _tc_body,
        grid=(_B // _NB,),
        in_specs=[pl.BlockSpec((_NB, _C, _H, _W), lambda b: (b, 0, 0, 0))],
        out_specs=[
            pl.BlockSpec((_NB, _C, _H, _W), lambda b: (b, 0, 0, 0)),
            pl.BlockSpec((_NB, 1, _L), lambda b: (b, 0, 0)),
        ],
        out_shape=[
            jax.ShapeDtypeStruct((_B, _C, _H, _W), jnp.float32),
            jax.ShapeDtypeStruct((_B, 1, _L), jnp.float32),
        ],
    )(grid)


def _fixup_planes(fbg, ftgt, bgval):
    def body(h, carry):
        for k in range(_W // _L):
            g = fbg[h, pl.ds(k * _L, _L)]
            t = ftgt[h, pl.ds(k * _L, _L)]
            m = g > 0.5
            fbg[h, pl.ds(k * _L, _L)] = jnp.where(m, bgval, g)
            ftgt[h, pl.ds(k * _L, _L)] = jnp.where(m, 1.0, t)
        return carry

    lax.fori_loop(0, _H, body, 0)


def _sc_body(g_ref, bg_ref, t_ref, out_ref,
             fbg0, ftgt0, fbg1, ftgt1, ibuf, tbuf,
             gs_a0, gs_b0, gs_a1, gs_b1, ws_a0, ws_b0, ws_a1, ws_b1):
    cid = lax.axis_index("c")
    sid = lax.axis_index("s")
    wid = sid * _NC + cid
    pltpu.sync_copy(t_ref, tbuf)
    tgt = jnp.max(tbuf[...]).astype(jnp.int32)

    fbg = (fbg0, fbg1)
    ftgt = (ftgt0, ftgt1)
    gsem = ((gs_a0, gs_b0), (gs_a1, gs_b1))
    wsem = ((ws_a0, ws_b0), (ws_a1, ws_b1))

    def issue_gathers(j):
        b = wid * _BPW + j
        p = j % 2
        pltpu.sync_copy(bg_ref.at[b, 0], ibuf)
        bgc = jnp.max(ibuf[...]).astype(jnp.int32)
        gb = pltpu.async_copy(g_ref.at[b, bgc], fbg[p], gsem[p][0])
        gt = pltpu.async_copy(g_ref.at[b, tgt], ftgt[p], gsem[p][1])
        return (b, bgc, gb, gt)

    pend = [None] * _BPW
    writes = [None] * _BPW
    pend[0] = issue_gathers(0)
    for j in range(_BPW):
        p = j % 2
        if j + 1 < _BPW:
            if j >= 1:
                writes[j - 1][0].wait()
                writes[j - 1][1].wait()
            pend[j + 1] = issue_gathers(j + 1)
        b, bgc, gb, gt = pend[j]
        gb.wait()
        gt.wait()
        bgval = jnp.where(bgc == tgt, 1.0, 0.0)
        _fixup_planes(fbg[p], ftgt[p], bgval)
        wb = pltpu.async_copy(fbg[p], out_ref.at[b, bgc], wsem[p][0])
        wt = pltpu.async_copy(ftgt[p], out_ref.at[b, tgt], wsem[p][1])
        writes[j] = (wb, wt)
    writes[_BPW - 2][0].wait()
    writes[_BPW - 2][1].wait()
    writes[_BPW - 1][0].wait()
    writes[_BPW - 1][1].wait()


def _sc_fixup(grid, bgv, tgt16, out_ref):
    sck = pl.kernel(
        _sc_body,
        out_type=(),
        mesh=_sc_mesh(),
        compiler_params=pltpu.CompilerParams(needs_layout_passes=False),
        scratch_types=[
            pltpu.VMEM((_H, _W), jnp.float32),
            pltpu.VMEM((_H, _W), jnp.float32),
            pltpu.VMEM((_H, _W), jnp.float32),
            pltpu.VMEM((_H, _W), jnp.float32),
            pltpu.VMEM((_L,), jnp.float32),
            pltpu.VMEM((_L,), jnp.float32),
            pltpu.SemaphoreType.DMA,
            pltpu.SemaphoreType.DMA,
            pltpu.SemaphoreType.DMA,
            pltpu.SemaphoreType.DMA,
            pltpu.SemaphoreType.DMA,
            pltpu.SemaphoreType.DMA,
            pltpu.SemaphoreType.DMA,
            pltpu.SemaphoreType.DMA,
        ],
    )
    sck(grid, bgv, tgt16, out_ref)


def kernel(grid, target_color):
    tgt16 = jnp.full((_L,), target_color, jnp.float32)
    out, bgv = _tc_copy_sums(grid)
    out_ref = jax.new_ref(out)
    _sc_fixup(grid, bgv, tgt16, out_ref)
    return out_ref[...]
